# Initial kernel scaffold; baseline (speedup 1.0000x reference)
import jax
import jax.numpy as jnp
from jax.experimental import pallas as pl


def kernel(x, embed_weight):
    raise NotImplementedError("write your pallas kernel here")


# speedup vs baseline: 5.3321x; 5.3321x over previous
import functools

import jax
import jax.numpy as jnp
from jax import lax
from jax.experimental import pallas as pl
from jax.experimental.pallas import tpu as pltpu
from jax.experimental.pallas import tpu_sc as plsc

_NC = 2
_NS = 16
_NW = _NC * _NS

_KI = 8
_CHUNK = _KI * 128


def _make_gather(num_rows: int, embed_dim: int):
    assert num_rows % (_NW * _CHUNK) == 0
    rows_per_w = num_rows // _NW
    steps = rows_per_w // _CHUNK
    irows_per_w = rows_per_w // 128

    mesh = plsc.VectorSubcoreMesh(core_axis_name="c", subcore_axis_name="s")

    @functools.partial(
        pl.kernel,
        mesh=mesh,
        out_type=jax.ShapeDtypeStruct((num_rows, embed_dim), jnp.float32),
        scratch_types=[
            pltpu.VMEM((_KI, 128), jnp.int32),
            pltpu.VMEM((_CHUNK, embed_dim), jnp.float32),
            pltpu.SemaphoreType.DMA,
        ],
        compiler_params=pltpu.CompilerParams(use_tc_tiling_on_sc=False),
    )
    def gather_kernel(ids_hbm, table_hbm, out_hbm, idx_v, rows_v, sem):
        wid = lax.axis_index("s") * _NC + lax.axis_index("c")
        irow_base = wid * irows_per_w

        def step(i, carry):
            r0 = irow_base + i * _KI
            pltpu.sync_copy(ids_hbm.at[pl.ds(r0, _KI)], idx_v)
            copies = [
                pltpu.async_copy(
                    table_hbm.at[idx_v.at[j]],
                    rows_v.at[pl.ds(j * 128, 128)],
                    sem,
                )
                for j in range(_KI)
            ]
            for c in copies:
                c.wait()
            pltpu.sync_copy(rows_v, out_hbm.at[pl.ds(r0 * 128, _CHUNK)])
            return carry

        lax.fori_loop(0, steps, step, 0)

    return gather_kernel


def kernel(x, embed_weight):
    batch, seq, feat = x.shape
    num_rows = batch * seq
    embed_dim = embed_weight.shape[1]

    ids = x[..., 0].astype(jnp.int32).reshape(num_rows // 128, 128)
    emb = _make_gather(num_rows, embed_dim)(ids, embed_weight)
    emb = emb.reshape(batch, seq, embed_dim)
    return jnp.concatenate([emb, x[..., 1:]], axis=-1)
